# baseline (speedup 1.0000x reference)
import jax
import jax.numpy as jnp
from jax import lax
from jax.experimental import pallas as pl
from jax.experimental.pallas import tpu as pltpu
from jax.experimental.pallas import tpu_sc as plsc

B = 16384
F = 26
V = 100000
E = 32
DD = 13
H1 = 256
H2 = 128

NC = 2
NS = 16
NW = NC * NS
L = 16

TASKS = F * E
TASKS_PER_W = TASKS // NW
CH = 4096
NCH = B // CH
NIN = CH // L
IA = 4
SB = 2
HV = 50048
HV2 = V - HV


def _gather_lo(buf, base, ids_v, rowA):
    def lo_val(idx):
        g = plsc.load_gather(rowA, [jnp.minimum(idx, HV - 1)])
        return jnp.where(idx < HV, g, 0.0)

    idxq = [ids_v[pl.ds(base + k * L, L)] for k in range(IA)]
    valsq = [lo_val(idxq[k]) for k in range(SB)]
    idxq = idxq[SB:]

    def inner(i, carry):
        vq, iq = carry
        buf[pl.ds(i * L, L)] = vq[0]
        vals_n = lo_val(iq[0])
        idx_n = ids_v[pl.ds(base + (i + IA) * L, L)]
        return (vq[1:] + (vals_n,), iq[1:] + (idx_n,))

    vq, _ = lax.fori_loop(0, NIN - SB, inner, (tuple(valsq), tuple(idxq)),
                          unroll=8)
    for k in range(SB):
        buf[pl.ds((NIN - SB + k) * L, L)] = vq[k]


def _gather_hi(buf, base, ids_v, rowB):
    def hi_val(idx):
        g = plsc.load_gather(rowB, [jnp.maximum(idx - HV, 0)])
        return jnp.where(idx >= HV, g, 0.0)

    idxq = [ids_v[pl.ds(base + k * L, L)] for k in range(IA)]
    valsq = [hi_val(idxq[k]) for k in range(SB)]
    idxq = idxq[SB:]

    def inner(j, carry):
        vq, iq = carry
        plsc.addupdate(buf.at[pl.ds(j * L, L)], vq[0])
        vals_n = hi_val(iq[0])
        idx_n = ids_v[pl.ds(base + (j + IA) * L, L)]
        return (vq[1:] + (vals_n,), iq[1:] + (idx_n,))

    vq, _ = lax.fori_loop(0, NIN - SB, inner, (tuple(valsq), tuple(idxq)),
                          unroll=8)
    for k in range(SB):
        plsc.addupdate(buf.at[pl.ds((NIN - SB + k) * L, L)], vq[k])


def _sc_gather_body(ids_hbm, table_hbm, out_hbm, rowA, rowB, ids_v,
                    out0_v, out1_v, semA, semB, sem0, sem1):
    wid = lax.axis_index("s") * NC + lax.axis_index("c")
    t0 = wid * TASKS_PER_W
    outs = (out0_v, out1_v)
    osems = (sem0, sem1)

    def wait_out(b, t):
        pltpu.make_async_copy(
            outs[b], out_hbm.at[t, pl.ds(0, CH)], osems[b]).wait()

    pltpu.async_copy(table_hbm.at[t0].at[pl.ds(0, HV)], rowA, semA)
    pltpu.async_copy(table_hbm.at[t0].at[pl.ds(HV, HV2)], rowB, semB)

    def task_body(ti, prev_f):
        t = t0 + ti
        f = t // E

        @pl.when(f != prev_f)
        def _():
            pltpu.sync_copy(ids_hbm.at[f], ids_v.at[pl.ds(0, B)])

        pltpu.make_async_copy(table_hbm.at[t].at[pl.ds(0, HV)], rowA, semA).wait()

        for c in (0, 1):
            @pl.when(ti > 0)
            def _(c=c):
                wait_out(c % 2, t)
            _gather_lo(outs[c % 2], c * CH, ids_v, rowA)
        pltpu.make_async_copy(table_hbm.at[t].at[pl.ds(HV, HV2)], rowB, semB).wait()
        for c in (0, 1):
            _gather_hi(outs[c % 2], c * CH, ids_v, rowB)
            pltpu.async_copy(outs[c % 2], out_hbm.at[t, pl.ds(c * CH, CH)],
                             osems[c % 2])
        for c in (2, 3):
            wait_out(c % 2, t)
            _gather_lo(outs[c % 2], c * CH, ids_v, rowA)

        @pl.when(ti + 1 < TASKS_PER_W)
        def _():
            pltpu.async_copy(table_hbm.at[t + 1].at[pl.ds(0, HV)], rowA, semA)

        for c in (2, 3):
            _gather_hi(outs[c % 2], c * CH, ids_v, rowB)
            pltpu.async_copy(outs[c % 2], out_hbm.at[t, pl.ds(c * CH, CH)],
                             osems[c % 2])

        @pl.when(ti + 1 < TASKS_PER_W)
        def _():
            pltpu.async_copy(table_hbm.at[t + 1].at[pl.ds(HV, HV2)], rowB, semB)

        return f

    lax.fori_loop(0, TASKS_PER_W, task_body, jnp.int32(-1))
    wait_out(0, t0)
    wait_out(1, t0)


def _sc_gather(ids_t, table2):
    mesh = plsc.VectorSubcoreMesh(
        core_axis_name="c", subcore_axis_name="s", num_cores=NC, num_subcores=NS
    )
    return pl.kernel(
        _sc_gather_body,
        out_type=jax.ShapeDtypeStruct((TASKS, B), jnp.float32),
        mesh=mesh,
        scratch_types=[
            pltpu.VMEM((HV,), jnp.float32),
            pltpu.VMEM((HV2,), jnp.float32),
            pltpu.VMEM((B + 2 * L,), jnp.int32),
            pltpu.VMEM((CH,), jnp.float32),
            pltpu.VMEM((CH,), jnp.float32),
            pltpu.SemaphoreType.DMA,
            pltpu.SemaphoreType.DMA,
            pltpu.SemaphoreType.DMA,
            pltpu.SemaphoreType.DMA,
        ],
        compiler_params=pltpu.CompilerParams(needs_layout_passes=False),
    )(ids_t, table2)


def _mlp_body(embt_ref, denset_ref, sew_ref, w1_ref, b1_ref, w2_ref,
              b2_ref, wf_ref, bf_ref, out_ref):
    embt = embt_ref[...]
    denset = denset_ref[...]
    dn = (((0,), (0,)), ((), ()))

    ri = lax.broadcasted_iota(jnp.int32, (F, F * E), 0)
    ci = lax.broadcasted_iota(jnp.int32, (F, F * E), 1) // E
    sel = (ri == ci).astype(jnp.float32)
    Z = jnp.dot(sel, embt, preferred_element_type=jnp.float32) * (1.0 / E)
    S = lax.dot_general(sew_ref[...], Z, dn, preferred_element_type=jnp.float32)
    S = S - jnp.max(S, axis=0, keepdims=True)
    Ex = jnp.exp(S)
    A = Ex / jnp.sum(Ex, axis=0, keepdims=True)
    Aexp = lax.dot_general(sel, A, dn, preferred_element_type=jnp.float32)
    se = embt * Aexp

    h = lax.dot_general(w1_ref[0:F * E, :], se, dn,
                        preferred_element_type=jnp.float32)
    h = h + lax.dot_general(w1_ref[F * E:, :], denset, dn,
                            preferred_element_type=jnp.float32)
    h = jnp.maximum(h + b1_ref[...], 0.0)
    h = jnp.maximum(
        lax.dot_general(w2_ref[...], h, dn, preferred_element_type=jnp.float32)
        + b2_ref[...], 0.0)
    out_ref[...] = (
        lax.dot_general(wf_ref[...], h, dn, preferred_element_type=jnp.float32)
        + bf_ref[...])


def _mlp(emb_t, dense_t, se_W, W1, b1, W2, b2, Wf, bf, bs=2048):
    grid = (B // bs,)
    return pl.pallas_call(
        _mlp_body,
        grid=grid,
        in_specs=[
            pl.BlockSpec((F * E, bs), lambda i: (0, i)),
            pl.BlockSpec((DD, bs), lambda i: (0, i)),
            pl.BlockSpec((F, F), lambda i: (0, 0)),
            pl.BlockSpec((F * E + DD, H1), lambda i: (0, 0)),
            pl.BlockSpec((H1, 1), lambda i: (0, 0)),
            pl.BlockSpec((H1, H2), lambda i: (0, 0)),
            pl.BlockSpec((H2, 1), lambda i: (0, 0)),
            pl.BlockSpec((H2, 1), lambda i: (0, 0)),
            pl.BlockSpec((1, 1), lambda i: (0, 0)),
        ],
        out_specs=pl.BlockSpec((1, bs), lambda i: (0, i)),
        out_shape=jax.ShapeDtypeStruct((1, B), jnp.float32),
    )(emb_t, dense_t, se_W, W1, b1, W2, b2, Wf, bf)


def kernel(sparse_ids, dense_vals, tables, se_W, W1, b1, W2, b2, Wf, bf):
    ids_t = sparse_ids.astype(jnp.int32).T
    table2 = tables.transpose(0, 2, 1).reshape(F * E, V)
    emb_t = _sc_gather(ids_t, table2)
    dense_t = dense_vals.T
    out_t = _mlp(
        emb_t,
        dense_t,
        se_W,
        W1,
        b1.reshape(H1, 1),
        W2,
        b2.reshape(H2, 1),
        Wf,
        bf.reshape(1, 1),
    )
    return out_t.reshape(B, 1)


# speedup vs baseline: 1.1919x; 1.1919x over previous
import jax
import jax.numpy as jnp
from jax import lax
from jax.experimental import pallas as pl
from jax.experimental.pallas import tpu as pltpu
from jax.experimental.pallas import tpu_sc as plsc

B = 16384
F = 26
V = 100000
E = 32
DD = 13
H1 = 256
H2 = 128

NC = 2
NS = 16
NW = NC * NS
L = 16

TASKS = F * E
TASKS_PER_W = TASKS // NW
CH = 4096
NCH = B // CH
NIN = CH // L
IA = 6
SB = 2


def _sc_gather_body(ids_hbm, table_hbm, out_hbm, row_v, ids_v, out0_v, out1_v,
                    sem0, sem1, sem_row):
    wid = lax.axis_index("s") * NC + lax.axis_index("c")
    outs = (out0_v, out1_v)
    sems = (sem0, sem1)

    def task_body(ti, prev_f):
        t = wid * TASKS_PER_W + ti
        f = t // E
        row_cp = pltpu.async_copy(table_hbm.at[t], row_v, sem_row)

        @pl.when(f != prev_f)
        def _():
            pltpu.sync_copy(ids_hbm.at[f], ids_v.at[pl.ds(0, B)])

        row_cp.wait()

        out_cps = []
        for c in range(NCH):
            buf, sem = outs[c % 2], sems[c % 2]
            if c >= 2:
                out_cps[c - 2].wait()

            base = c * CH
            idxq = [ids_v[pl.ds(base + k * L, L)] for k in range(IA)]
            valsq = [plsc.load_gather(row_v, [idxq[k]]) for k in range(SB)]
            idxq = idxq[SB:]

            def inner(i, carry, buf=buf, base=base):
                vq, iq = carry
                buf[pl.ds(i * L, L)] = vq[0]
                vals_n = plsc.load_gather(row_v, [iq[0]])
                idx_n = ids_v[pl.ds(base + (i + IA) * L, L)]
                return (vq[1:] + (vals_n,), iq[1:] + (idx_n,))

            vq, _ = lax.fori_loop(0, NIN - SB, inner,
                                  (tuple(valsq), tuple(idxq)), unroll=16)
            for k in range(SB):
                buf[pl.ds((NIN - SB + k) * L, L)] = vq[k]
            out_cps.append(
                pltpu.async_copy(buf, out_hbm.at[t, pl.ds(c * CH, CH)], sem))
        out_cps[-2].wait()
        out_cps[-1].wait()
        return f

    lax.fori_loop(0, TASKS_PER_W, task_body, jnp.int32(-1))


def _sc_gather(ids_t, table2):
    mesh = plsc.VectorSubcoreMesh(
        core_axis_name="c", subcore_axis_name="s", num_cores=NC, num_subcores=NS
    )
    return pl.kernel(
        _sc_gather_body,
        out_type=jax.ShapeDtypeStruct((TASKS, B), jnp.float32),
        mesh=mesh,
        scratch_types=[
            pltpu.VMEM((V,), jnp.float32),
            pltpu.VMEM((B + (IA - SB) * L,), jnp.int32),
            pltpu.VMEM((CH,), jnp.float32),
            pltpu.VMEM((CH,), jnp.float32),
            pltpu.SemaphoreType.DMA,
            pltpu.SemaphoreType.DMA,
            pltpu.SemaphoreType.DMA,
        ],
        compiler_params=pltpu.CompilerParams(needs_layout_passes=False),
    )(ids_t, table2)


def _mlp_body(embt_ref, denset_ref, sew_ref, w1_ref, b1_ref, w2_ref,
              b2_ref, wf_ref, bf_ref, out_ref):
    embt = embt_ref[...]
    denset = denset_ref[...]
    dn = (((0,), (0,)), ((), ()))

    ri = lax.broadcasted_iota(jnp.int32, (F, F * E), 0)
    ci = lax.broadcasted_iota(jnp.int32, (F, F * E), 1) // E
    sel = (ri == ci).astype(jnp.float32)
    Z = jnp.dot(sel, embt, preferred_element_type=jnp.float32) * (1.0 / E)
    S = lax.dot_general(sew_ref[...], Z, dn, preferred_element_type=jnp.float32)
    S = S - jnp.max(S, axis=0, keepdims=True)
    Ex = jnp.exp(S)
    A = Ex / jnp.sum(Ex, axis=0, keepdims=True)
    Aexp = lax.dot_general(sel, A, dn, preferred_element_type=jnp.float32)
    se = embt * Aexp

    h = lax.dot_general(w1_ref[0:F * E, :], se, dn,
                        preferred_element_type=jnp.float32)
    h = h + lax.dot_general(w1_ref[F * E:, :], denset, dn,
                            preferred_element_type=jnp.float32)
    h = jnp.maximum(h + b1_ref[...], 0.0)
    h = jnp.maximum(
        lax.dot_general(w2_ref[...], h, dn, preferred_element_type=jnp.float32)
        + b2_ref[...], 0.0)
    out_ref[...] = (
        lax.dot_general(wf_ref[...], h, dn, preferred_element_type=jnp.float32)
        + bf_ref[...])


def _mlp(emb_t, dense_t, se_W, W1, b1, W2, b2, Wf, bf, bs=2048):
    grid = (B // bs,)
    return pl.pallas_call(
        _mlp_body,
        grid=grid,
        in_specs=[
            pl.BlockSpec((F * E, bs), lambda i: (0, i)),
            pl.BlockSpec((DD, bs), lambda i: (0, i)),
            pl.BlockSpec((F, F), lambda i: (0, 0)),
            pl.BlockSpec((F * E + DD, H1), lambda i: (0, 0)),
            pl.BlockSpec((H1, 1), lambda i: (0, 0)),
            pl.BlockSpec((H1, H2), lambda i: (0, 0)),
            pl.BlockSpec((H2, 1), lambda i: (0, 0)),
            pl.BlockSpec((H2, 1), lambda i: (0, 0)),
            pl.BlockSpec((1, 1), lambda i: (0, 0)),
        ],
        out_specs=pl.BlockSpec((1, bs), lambda i: (0, i)),
        out_shape=jax.ShapeDtypeStruct((1, B), jnp.float32),
    )(emb_t, dense_t, se_W, W1, b1, W2, b2, Wf, bf)


def kernel(sparse_ids, dense_vals, tables, se_W, W1, b1, W2, b2, Wf, bf):
    ids_t = sparse_ids.astype(jnp.int32).T
    table2 = tables.transpose(0, 2, 1).reshape(F * E, V)
    emb_t = _sc_gather(ids_t, table2)
    dense_t = dense_vals.T
    out_t = _mlp(
        emb_t,
        dense_t,
        se_W,
        W1,
        b1.reshape(H1, 1),
        W2,
        b2.reshape(H2, 1),
        Wf,
        bf.reshape(1, 1),
    )
    return out_t.reshape(B, 1)


# speedup vs baseline: 1.1950x; 1.0027x over previous
import jax
import jax.numpy as jnp
from jax import lax
from jax.experimental import pallas as pl
from jax.experimental.pallas import tpu as pltpu
from jax.experimental.pallas import tpu_sc as plsc

B = 16384
F = 26
V = 100000
E = 32
DD = 13
H1 = 256
H2 = 128

NC = 2
NS = 16
NW = NC * NS
L = 16

TASKS = F * E
TASKS_PER_W = TASKS // NW
CH = 4096
NCH = B // CH
NIN = CH // L
IA = 6
SB = 2


def _sc_gather_body(ids_hbm, table_hbm, out_hbm, row_v, ids_v, out0_v, out1_v,
                    sem0, sem1, sem_row):
    wid = lax.axis_index("s") * NC + lax.axis_index("c")
    outs = (out0_v, out1_v)
    sems = (sem0, sem1)

    def task_body(ti, prev_f):
        t = wid * TASKS_PER_W + ti
        f = t // E
        row_cp = pltpu.async_copy(table_hbm.at[t], row_v, sem_row)

        @pl.when(f != prev_f)
        def _():
            pltpu.sync_copy(ids_hbm.at[f], ids_v.at[pl.ds(0, B)])

        row_cp.wait()

        out_cps = []
        for c in range(NCH):
            buf, sem = outs[c % 2], sems[c % 2]
            if c >= 2:
                out_cps[c - 2].wait()

            base = c * CH
            idxq = [ids_v[pl.ds(base + k * L, L)] for k in range(IA)]
            valsq = [plsc.load_gather(row_v, [idxq[k]]) for k in range(SB)]
            idxq = idxq[SB:]

            def inner(i, carry, buf=buf, base=base):
                vq, iq = carry
                buf[pl.ds(i * L, L)] = vq[0]
                vals_n = plsc.load_gather(row_v, [iq[0]])
                idx_n = ids_v[pl.ds(base + (i + IA) * L, L)]
                return (vq[1:] + (vals_n,), iq[1:] + (idx_n,))

            vq, _ = lax.fori_loop(0, NIN - SB, inner,
                                  (tuple(valsq), tuple(idxq)), unroll=16)
            for k in range(SB):
                buf[pl.ds((NIN - SB + k) * L, L)] = vq[k]
            out_cps.append(
                pltpu.async_copy(buf, out_hbm.at[t, pl.ds(c * CH, CH)], sem))
        out_cps[-2].wait()
        out_cps[-1].wait()
        return f

    lax.fori_loop(0, TASKS_PER_W, task_body, jnp.int32(-1))


def _sc_gather(ids_t, table2):
    mesh = plsc.VectorSubcoreMesh(
        core_axis_name="c", subcore_axis_name="s", num_cores=NC, num_subcores=NS
    )
    return pl.kernel(
        _sc_gather_body,
        out_type=jax.ShapeDtypeStruct((TASKS, B), jnp.float32),
        mesh=mesh,
        scratch_types=[
            pltpu.VMEM((V,), jnp.float32),
            pltpu.VMEM((B + (IA - SB) * L,), jnp.int32),
            pltpu.VMEM((CH,), jnp.float32),
            pltpu.VMEM((CH,), jnp.float32),
            pltpu.SemaphoreType.DMA,
            pltpu.SemaphoreType.DMA,
            pltpu.SemaphoreType.DMA,
        ],
        compiler_params=pltpu.CompilerParams(needs_layout_passes=False),
    )(ids_t, table2)


def _mlp_body(embt_ref, denset_ref, sew_ref, w1_ref, b1_ref, w2_ref,
              b2_ref, wf_ref, bf_ref, out_ref):
    embt = embt_ref[...]
    denset = denset_ref[...]
    dn = (((0,), (0,)), ((), ()))

    ri = lax.broadcasted_iota(jnp.int32, (F, F * E), 0)
    ci = lax.broadcasted_iota(jnp.int32, (F, F * E), 1) // E
    sel = (ri == ci).astype(jnp.float32)
    Z = jnp.dot(sel, embt, preferred_element_type=jnp.float32) * (1.0 / E)
    S = lax.dot_general(sew_ref[...], Z, dn, preferred_element_type=jnp.float32)
    S = S - jnp.max(S, axis=0, keepdims=True)
    Ex = jnp.exp(S)
    A = Ex / jnp.sum(Ex, axis=0, keepdims=True)
    Aexp = lax.dot_general(sel, A, dn, preferred_element_type=jnp.float32)
    se = embt * Aexp

    h = lax.dot_general(w1_ref[0:F * E, :], se, dn,
                        preferred_element_type=jnp.float32)
    h = h + lax.dot_general(w1_ref[F * E:, :], denset, dn,
                            preferred_element_type=jnp.float32)
    h = jnp.maximum(h + b1_ref[...], 0.0)
    h = jnp.maximum(
        lax.dot_general(w2_ref[...], h, dn, preferred_element_type=jnp.float32)
        + b2_ref[...], 0.0)
    out_ref[...] = (
        lax.dot_general(wf_ref[...], h, dn, preferred_element_type=jnp.float32)
        + bf_ref[...])


def _mlp(emb_t, dense_t, se_W, W1, b1, W2, b2, Wf, bf, bs=4096):
    grid = (B // bs,)
    return pl.pallas_call(
        _mlp_body,
        grid=grid,
        in_specs=[
            pl.BlockSpec((F * E, bs), lambda i: (0, i)),
            pl.BlockSpec((DD, bs), lambda i: (0, i)),
            pl.BlockSpec((F, F), lambda i: (0, 0)),
            pl.BlockSpec((F * E + DD, H1), lambda i: (0, 0)),
            pl.BlockSpec((H1, 1), lambda i: (0, 0)),
            pl.BlockSpec((H1, H2), lambda i: (0, 0)),
            pl.BlockSpec((H2, 1), lambda i: (0, 0)),
            pl.BlockSpec((H2, 1), lambda i: (0, 0)),
            pl.BlockSpec((1, 1), lambda i: (0, 0)),
        ],
        out_specs=pl.BlockSpec((1, bs), lambda i: (0, i)),
        out_shape=jax.ShapeDtypeStruct((1, B), jnp.float32),
    )(emb_t, dense_t, se_W, W1, b1, W2, b2, Wf, bf)


def kernel(sparse_ids, dense_vals, tables, se_W, W1, b1, W2, b2, Wf, bf):
    ids_t = sparse_ids.astype(jnp.int32).T
    table2 = tables.transpose(0, 2, 1).reshape(F * E, V)
    emb_t = _sc_gather(ids_t, table2)
    dense_t = dense_vals.T
    out_t = _mlp(
        emb_t,
        dense_t,
        se_W,
        W1,
        b1.reshape(H1, 1),
        W2,
        b2.reshape(H2, 1),
        Wf,
        bf.reshape(1, 1),
    )
    return out_t.reshape(B, 1)
